# baseline (device time: 15574 ns/iter reference)
import jax
import jax.numpy as jnp
from jax import lax
from jax.experimental import pallas as pl
from jax.experimental.pallas import tpu as pltpu

HALF = 512
ROWS = 256
K = 8
CH = ROWS // K


def kernel(x):
    _, m, n = x.shape

    def body(x_ref, out_ref, a_ref, b_ref, asend, arecv, bsend, brecv):
        my_x = lax.axis_index("x")
        my_y = lax.axis_index("y")
        my_z = lax.axis_index("z")
        partner = (my_x, 1 - my_y, my_z)
        sibling = (my_x, my_y, 1 - my_z)

        barrier_sem = pltpu.get_barrier_semaphore()
        for nbr in (partner, sibling):
            pl.semaphore_signal(
                barrier_sem, inc=1, device_id=nbr,
                device_id_type=pl.DeviceIdType.MESH,
            )
        pl.semaphore_wait(barrier_sem, 2)

        row0 = my_z * ROWS
        orow0 = (1 - my_z) * ROWS
        col_mine = my_y * HALF
        col_partner = (1 - my_y) * HALF

        a = []
        for c in range(K):
            rd = pltpu.make_async_remote_copy(
                src_ref=x_ref.at[
                    0, pl.ds(row0 + c * CH, CH), pl.ds(col_partner, HALF)
                ],
                dst_ref=a_ref.at[pl.ds(c * CH, CH), :],
                send_sem=asend.at[c],
                recv_sem=arecv.at[c],
                device_id=partner,
                device_id_type=pl.DeviceIdType.MESH,
            )
            rd.start()
            a.append(rd)

        b = []
        for c in range(K):
            a[c].wait_recv()
            rd = pltpu.make_async_remote_copy(
                src_ref=a_ref.at[pl.ds(c * CH, CH), :],
                dst_ref=b_ref.at[pl.ds(c * CH, CH), :],
                send_sem=bsend.at[c],
                recv_sem=brecv.at[c],
                device_id=sibling,
                device_id_type=pl.DeviceIdType.MESH,
            )
            rd.start()
            b.append(rd)

        out_ref[pl.ds(row0, ROWS), :] = (
            a_ref[:, :] + x_ref[0, pl.ds(row0, ROWS), pl.ds(col_mine, HALF)]
        )

        for c in range(K):
            b[c].wait_recv()
        out_ref[pl.ds(orow0, ROWS), :] = (
            b_ref[:, :] + x_ref[0, pl.ds(orow0, ROWS), pl.ds(col_mine, HALF)]
        )

        for c in range(K):
            a[c].wait_send()
            b[c].wait_send()

    return pl.pallas_call(
        body,
        out_shape=jax.ShapeDtypeStruct((m, HALF), x.dtype),
        in_specs=[pl.BlockSpec(memory_space=pltpu.VMEM)],
        out_specs=pl.BlockSpec(memory_space=pltpu.VMEM),
        scratch_shapes=[
            pltpu.VMEM((ROWS, HALF), x.dtype),
            pltpu.VMEM((ROWS, HALF), x.dtype),
            pltpu.SemaphoreType.DMA((K,)),
            pltpu.SemaphoreType.DMA((K,)),
            pltpu.SemaphoreType.DMA((K,)),
            pltpu.SemaphoreType.DMA((K,)),
        ],
        compiler_params=pltpu.CompilerParams(collective_id=0),
    )(x)
